# baseline (speedup 1.0000x reference)
import functools

import jax
import jax.numpy as jnp
from jax.experimental import pallas as pl
from jax.experimental.pallas import tpu as pltpu


def _round_up(v, m):
    return (v + m - 1) // m * m


def _mm_body(a_ref, b_ref, o_ref, *, nk, bm, bn, zero_diag):
    k = pl.program_id(2)

    @pl.when(k == 0)
    def _init():
        o_ref[...] = jnp.zeros_like(o_ref)

    o_ref[...] += jnp.dot(a_ref[...], b_ref[...],
                          preferred_element_type=jnp.float32)

    if zero_diag:
        @pl.when(k == nk - 1)
        def _mask():
            i = pl.program_id(0)
            j = pl.program_id(1)
            rows = i * bm + jax.lax.broadcasted_iota(jnp.int32, (bm, bn), 0)
            cols = j * bn + jax.lax.broadcasted_iota(jnp.int32, (bm, bn), 1)
            o_ref[...] = jnp.where(rows == cols, 0.0, o_ref[...])


def _matmul(a, b, zero_diag=False):
    m, k = a.shape
    k2, n = b.shape
    assert k == k2
    bm = 256 if m >= 256 else _round_up(m, 8)
    bn = 256 if n >= 256 else _round_up(n, 128)
    bk = 512 if k >= 512 else _round_up(k, 128)
    mp, kp, np_ = _round_up(m, bm), _round_up(k, bk), _round_up(n, bn)
    if (mp, kp) != (m, k):
        a = jnp.pad(a, ((0, mp - m), (0, kp - k)))
    if (kp, np_) != (k, n):
        b = jnp.pad(b, ((0, kp - k), (0, np_ - n)))
    nk = kp // bk
    out = pl.pallas_call(
        functools.partial(_mm_body, nk=nk, bm=bm, bn=bn, zero_diag=zero_diag),
        grid=(mp // bm, np_ // bn, nk),
        in_specs=[
            pl.BlockSpec((bm, bk), lambda i, j, kk: (i, kk)),
            pl.BlockSpec((bk, bn), lambda i, j, kk: (kk, j)),
        ],
        out_specs=pl.BlockSpec((bm, bn), lambda i, j, kk: (i, j)),
        out_shape=jax.ShapeDtypeStruct((mp, np_), jnp.float32),
        compiler_params=pltpu.CompilerParams(
            dimension_semantics=("parallel", "parallel", "arbitrary")),
    )(a, b)
    if (mp, np_) != (m, n):
        out = out[:m, :n]
    return out


def _gcn(A, x, W, b, deg):
    dinv = jnp.where(deg > 0.0, jax.lax.rsqrt(deg), 0.0)
    z = dinv[:, None] * _matmul(x, W)
    az = _matmul(A, z)
    return dinv[:, None] * (az + 2.0 * z) + b


def _pool_score(x, w):
    return jnp.tanh((x @ w) / jnp.linalg.norm(w))


def kernel(x, edge_index, W_down0, b_down0, W_down1, b_down1, W_down2,
           b_down2, W_down3, b_down3, p0, p1, p2, W_up0, b_up0, W_up1, b_up1,
           W_up2, b_up2):
    n0 = x.shape[0]
    relu = jax.nn.relu
    src = edge_index[0]
    dst = edge_index[1]

    A0 = jnp.zeros((n0, n0), jnp.float32).at[dst, src].add(1.0)
    deg0 = jnp.bincount(dst, length=n0).astype(jnp.float32) + 2.0

    x0 = relu(_gcn(A0, x, W_down0, b_down0, deg0))

    ratio = 0.5
    pvecs = (p0, p1, p2)
    W_downs = (W_down1, W_down2, W_down3)
    b_downs = (b_down1, b_down2, b_down3)

    xs = [x0]
    As = [A0]
    degs = [deg0]
    perms = []
    xcur = x0
    Acur = A0
    n = n0
    for lvl in range(3):
        k = -(-n // 2)
        score = _pool_score(xcur, pvecs[lvl])
        vals, perm = jax.lax.top_k(score, k)
        if lvl == 0:
            Ai = jnp.fill_diagonal(Acur, 1.0, inplace=False)
            B = Ai[perm, :]
            C = Ai[:, perm]
        else:
            ar = jnp.arange(k)
            B = Acur[perm, :].at[ar, perm].add(1.0)
            C = Acur[:, perm].at[perm, ar].add(1.0)
        Ap = _matmul(B, C, zero_diag=True)
        degp = Ap.sum(axis=1) + 2.0
        xp = xcur[perm] * vals[:, None]
        xcur = relu(_gcn(Ap, xp, W_downs[lvl], b_downs[lvl], degp))
        perms.append(perm)
        if lvl < 2:
            xs.append(xcur)
            As.append(Ap)
            degs.append(degp)
        Acur = Ap
        n = k

    W_ups = (W_up0, W_up1, W_up2)
    b_ups = (b_up0, b_up1, b_up2)
    for i in range(3):
        j = 2 - i
        res = xs[j]
        perm = perms[j]
        up = jnp.zeros_like(res).at[perm].set(xcur)
        xcur = _gcn(As[j], res + up, W_ups[i], b_ups[i], degs[j])
        if i < 2:
            xcur = relu(xcur)
    return xcur


# speedup vs baseline: 1.9881x; 1.9881x over previous
import functools

import jax
import jax.numpy as jnp
from jax.experimental import pallas as pl
from jax.experimental.pallas import tpu as pltpu

_OOB = jnp.int32(1 << 30)


def _round_up(v, m):
    return (v + m - 1) // m * m


def _pick(d, opts):
    for o in opts:
        if d % o == 0:
            return o
    return None


def _mm_body(a_ref, b_ref, o_ref, *, nk, bm, bn, zero_diag):
    k = pl.program_id(2)

    @pl.when(k == 0)
    def _init():
        o_ref[...] = jnp.zeros_like(o_ref)

    o_ref[...] += jnp.dot(a_ref[...], b_ref[...],
                          preferred_element_type=jnp.float32)

    if zero_diag:
        @pl.when(k == nk - 1)
        def _mask():
            i = pl.program_id(0)
            j = pl.program_id(1)
            rows = i * bm + jax.lax.broadcasted_iota(jnp.int32, (bm, bn), 0)
            cols = j * bn + jax.lax.broadcasted_iota(jnp.int32, (bm, bn), 1)
            o_ref[...] = jnp.where(rows == cols, 0.0, o_ref[...])


def _matmul(a, b, zero_diag=False):
    m, k = a.shape
    k2, n = b.shape
    assert k == k2
    bm = _pick(m, (1280, 512, 256, 8))
    bn = _pick(n, (2560, 1280, 512, 128))
    bk = _pick(k, (640, 128))
    nk = k // bk
    return pl.pallas_call(
        functools.partial(_mm_body, nk=nk, bm=bm, bn=bn, zero_diag=zero_diag),
        grid=(m // bm, n // bn, nk),
        in_specs=[
            pl.BlockSpec((bm, bk), lambda i, j, kk: (i, kk)),
            pl.BlockSpec((bk, bn), lambda i, j, kk: (kk, j)),
        ],
        out_specs=pl.BlockSpec((bm, bn), lambda i, j, kk: (i, j)),
        out_shape=jax.ShapeDtypeStruct((m, n), jnp.float32),
        compiler_params=pltpu.CompilerParams(
            dimension_semantics=("parallel", "parallel", "arbitrary")),
    )(a, b)


def _gcn(A, x, W, b, deg, diag=None):
    dinv = jnp.where(deg > 0.0, jax.lax.rsqrt(deg), 0.0)
    z = dinv[:, None] * _matmul(x, W)
    az = _matmul(A, z)
    d = 2.0 if diag is None else (diag + 2.0)[:, None]
    return dinv[:, None] * (az + d * z) + b


def kernel(x, edge_index, W_down0, b_down0, W_down1, b_down1, W_down2,
           b_down2, W_down3, b_down3, p0, p1, p2, W_up0, b_up0, W_up1, b_up1,
           W_up2, b_up2):
    n0 = x.shape[0]
    np0 = _round_up(n0, 1280)
    relu = jax.nn.relu
    src = edge_index[0]
    dst = edge_index[1]
    is_self = src == dst
    ones_e = jnp.ones_like(src, jnp.float32)

    dstm = jnp.where(is_self, _OOB, dst)
    A0 = jnp.zeros((np0, np0), jnp.float32).at[dstm, src].add(
        ones_e, mode='drop')
    self0 = jnp.zeros((np0,), jnp.float32).at[dst].add(
        is_self.astype(jnp.float32))
    deg0 = jnp.zeros((np0,), jnp.float32).at[dst].add(ones_e) + 2.0

    xp = jnp.zeros((np0, x.shape[1]), jnp.float32).at[:n0].set(x)
    x0 = relu(_gcn(A0, xp, W_down0, b_down0, deg0, diag=self0))

    pvecs = (p0, p1, p2)
    W_downs = (W_down1, W_down2, W_down3)
    b_downs = (b_down1, b_down2, b_down3)

    xs = [x0]
    As = [A0]
    degs = [deg0]
    diags = [self0]
    perms = []
    ns = [n0]
    xcur = x0
    Acur = A0
    n, npad = n0, np0
    for lvl in range(3):
        k = -(-n // 2)
        kp = _round_up(k, 1280)
        score = jnp.tanh((xcur[:n] @ pvecs[lvl]) /
                         jnp.linalg.norm(pvecs[lvl]))
        vals, perm = jax.lax.top_k(score, k)
        ar = jnp.arange(k)
        perm_p = jnp.concatenate([perm, jnp.full((kp - k,), n, jnp.int32)])
        B = Acur[perm_p, :].at[ar, perm].add(1.0)
        if lvl == 0:
            rank = jnp.full((n0,), _OOB, jnp.int32).at[perm].set(ar)
            C = jnp.zeros((npad, kp), jnp.float32).at[dstm, rank[src]].add(
                ones_e, mode='drop')
            C = C.at[perm, ar].add(1.0)
        else:
            C = jnp.take(Acur, perm_p, axis=1).at[perm, ar].add(1.0)
        Ap = _matmul(B, C, zero_diag=True)
        degp = Ap.sum(axis=1) + 2.0
        vals_p = jnp.zeros((kp,), jnp.float32).at[:k].set(vals)
        xpool = xcur[perm_p] * vals_p[:, None]
        xcur = relu(_gcn(Ap, xpool, W_downs[lvl], b_downs[lvl], degp))
        perms.append(perm)
        if lvl < 2:
            xs.append(xcur)
            As.append(Ap)
            degs.append(degp)
            diags.append(None)
        Acur = Ap
        n, npad = k, kp
        ns.append(k)

    W_ups = (W_up0, W_up1, W_up2)
    b_ups = (b_up0, b_up1, b_up2)
    for i in range(3):
        j = 2 - i
        res = xs[j]
        perm = perms[j]
        k = ns[j + 1]
        up = jnp.zeros_like(res).at[perm].set(xcur[:k])
        Wu = W_ups[i]
        bu = b_ups[i]
        dout = Wu.shape[1]
        if dout % 128 != 0:
            dp = _round_up(dout, 128)
            Wu = jnp.zeros((Wu.shape[0], dp), jnp.float32).at[:, :dout].set(Wu)
            bu = jnp.zeros((dp,), jnp.float32).at[:dout].set(bu)
        xcur = _gcn(As[j], res + up, Wu, bu, degs[j], diag=diags[j])
        if i < 2:
            xcur = relu(xcur)
    return xcur[:n0, :W_up2.shape[1]]
